# Initial kernel scaffold; baseline (speedup 1.0000x reference)
import functools

import jax
import jax.numpy as jnp
from jax import lax
from jax.experimental import pallas as pl
from jax.experimental.pallas import tpu as pltpu
from jax.experimental.pallas import tpu_sc as plsc

N = 10000
E = 320000
D = 128
AVG_LIN = 32.0
AVG_LOG = 3.4965075810294595

NT = 32
NPT = 320
NPAD = NT * NPT
CAP = 12288
CE = 64
CHUNK = 8000
NCHUNKS = E // CHUNK
NGROUPS = CHUNK // 16
ACC_ROWS = NPT + 1
DUMMY = NPT
NEG = -3.0e38

_MESH = plsc.VectorSubcoreMesh(core_axis_name="c", subcore_axis_name="s")


def _wid():
    return lax.axis_index("s") * 2 + lax.axis_index("c")


@functools.partial(
    pl.kernel,
    out_type=(
        jax.ShapeDtypeStruct((NT, CAP), jnp.int32),
        jax.ShapeDtypeStruct((NT, CAP), jnp.int32),
        jax.ShapeDtypeStruct((NT, 16), jnp.int32),
    ),
    mesh=_MESH,
    scratch_types=[
        pltpu.VMEM((CHUNK,), jnp.int32),
        pltpu.VMEM((CHUNK,), jnp.int32),
        pltpu.VMEM((CAP + CE,), jnp.int32),
        pltpu.VMEM((CAP + CE,), jnp.int32),
        pltpu.VMEM((16,), jnp.int32),
    ],
)
def _sc_prep(src_hbm, dst_hbm, srcl_hbm, dstl_hbm, nedges_hbm,
             sbuf, dbuf, srcl_v, dstl_v, nbuf):
    t = _wid()
    lo = t * NPT
    hi = lo + NPT

    def chunk_body(c, wp_vec):
        pltpu.sync_copy(src_hbm.at[pl.ds(c * CHUNK, CHUNK)], sbuf)
        pltpu.sync_copy(dst_hbm.at[pl.ds(c * CHUNK, CHUNK)], dbuf)

        def group_body(g, wp_vec):
            dvec = dbuf[pl.ds(g * 16, 16)]
            svec = sbuf[pl.ds(g * 16, 16)]
            m = (dvec >= lo) & (dvec < hi)
            w0 = jnp.minimum(wp_vec[0], CAP)
            plsc.store_compressed(srcl_v.at[pl.ds(w0, 16)], svec, mask=m)
            plsc.store_compressed(dstl_v.at[pl.ds(w0, 16)], dvec - lo, mask=m)
            return wp_vec + plsc.all_reduce_population_count(m)

        return lax.fori_loop(0, NGROUPS, group_body, wp_vec)

    wp_vec = lax.fori_loop(0, NCHUNKS, chunk_body,
                           jnp.zeros((16,), jnp.int32))
    wp = jnp.minimum(wp_vec[0], CAP)
    zeros16 = jnp.zeros((16,), jnp.int32)
    dummy16 = jnp.full((16,), DUMMY, jnp.int32)
    for k in range(CE // 16):
        srcl_v[pl.ds(wp + k * 16, 16)] = zeros16
        dstl_v[pl.ds(wp + k * 16, 16)] = dummy16
    wp_pad = ((wp + CE - 1) // CE) * CE
    pltpu.sync_copy(srcl_v.at[pl.ds(0, CAP)], srcl_hbm.at[t])
    pltpu.sync_copy(dstl_v.at[pl.ds(0, CAP)], dstl_hbm.at[t])
    iota = lax.iota(jnp.int32, 16)
    nbuf[...] = jnp.where(iota == 0, wp_pad, 0)
    pltpu.sync_copy(nbuf, nedges_hbm.at[t])


def _make_sc_layer(with_count):
    outs = [
        jax.ShapeDtypeStruct((NPAD * D,), jnp.float32),
        jax.ShapeDtypeStruct((NPAD * D,), jnp.float32),
    ]
    if with_count:
        outs.append(jax.ShapeDtypeStruct((NPAD * 16,), jnp.float32))

    scratch = [
        pltpu.VMEM((CAP,), jnp.int32),
        pltpu.VMEM((CAP,), jnp.int32),
        pltpu.VMEM((CE, D), jnp.float32),
        pltpu.VMEM((CE, D), jnp.float32),
        pltpu.VMEM((ACC_ROWS * D,), jnp.float32),
        pltpu.VMEM((ACC_ROWS * D,), jnp.float32),
        pltpu.VMEM((ACC_ROWS * 16,), jnp.float32),
        pltpu.VMEM((16,), jnp.int32),
        pltpu.SemaphoreType.DMA,
        pltpu.SemaphoreType.DMA,
    ]

    def body(b_hbm, srcl_hbm, dstl_hbm, nedges_hbm, ss_hbm, sm_hbm, *rest):
        if with_count:
            cnt_hbm = rest[0]
            rest = rest[1:]
        srcl_v, dstl_v, r0, r1, acc_s, acc_m, acc_c, nbuf, sem0, sem1 = rest
        t = _wid()
        pltpu.sync_copy(srcl_hbm.at[t], srcl_v)
        pltpu.sync_copy(dstl_hbm.at[t], dstl_v)
        pltpu.sync_copy(nedges_hbm.at[t], nbuf)
        my_n = jnp.sum(nbuf[...])
        nchunks = my_n // CE

        zf = jnp.zeros((16,), jnp.float32)
        nf = jnp.full((16,), NEG, jnp.float32)

        def init_body(i, _):
            acc_s[pl.ds(i * 16, 16)] = zf
            acc_m[pl.ds(i * 16, 16)] = nf
            return 0
        lax.fori_loop(0, ACC_ROWS * D // 16, init_body, 0)
        if with_count:
            def initc_body(i, _):
                acc_c[pl.ds(i * 16, 16)] = zf
                return 0
            lax.fori_loop(0, ACC_ROWS, initc_body, 0)

        ones = jnp.ones((16,), jnp.float32)

        def process(rbuf, ebase):
            def edge_body(e, _):
                d = dstl_v[ebase + e]
                ab = d * D
                for f in range(D // 16):
                    r = rbuf[e, pl.ds(f * 16, 16)]
                    plsc.addupdate(acc_s.at[pl.ds(ab + f * 16, 16)], r)
                    mm = acc_m[pl.ds(ab + f * 16, 16)]
                    acc_m[pl.ds(ab + f * 16, 16)] = jnp.maximum(mm, r)
                if with_count:
                    plsc.addupdate(acc_c.at[pl.ds(d * 16, 16)], ones)
                return 0
            lax.fori_loop(0, CE, edge_body, 0)

        def issue(c, rbuf, sem):
            pltpu.async_copy(
                b_hbm.at[srcl_v.at[pl.ds(c * CE, CE)]], rbuf, sem)

        def wait(rbuf, sem):
            pltpu.make_async_copy(
                b_hbm.at[srcl_v.at[pl.ds(0, CE)]], rbuf, sem).wait()

        @pl.when(nchunks > 0)
        def _():
            issue(0, r0, sem0)

        def chunk_body(c, _):
            even = (c % 2) == 0

            @pl.when(even)
            def _():
                wait(r0, sem0)

                @pl.when(c + 1 < nchunks)
                def _():
                    issue(c + 1, r1, sem1)
                process(r0, c * CE)

            @pl.when(jnp.logical_not(even))
            def _():
                wait(r1, sem1)

                @pl.when(c + 1 < nchunks)
                def _():
                    issue(c + 1, r0, sem0)
                process(r1, c * CE)
            return 0

        lax.fori_loop(0, nchunks, chunk_body, 0)

        pltpu.sync_copy(acc_s.at[pl.ds(0, NPT * D)],
                        ss_hbm.at[pl.ds(t * NPT * D, NPT * D)])
        pltpu.sync_copy(acc_m.at[pl.ds(0, NPT * D)],
                        sm_hbm.at[pl.ds(t * NPT * D, NPT * D)])
        if with_count:
            pltpu.sync_copy(acc_c.at[pl.ds(0, NPT * 16)],
                            cnt_hbm.at[pl.ds(t * NPT * 16, NPT * 16)])

    return pl.kernel(body, out_type=tuple(outs), mesh=_MESH,
                     scratch_types=scratch)


_sc_layer_count = _make_sc_layer(True)
_sc_layer_plain = _make_sc_layer(False)


ROWS = 2000
GRID = N // ROWS


def _dot(a, b):
    return jnp.dot(a, b, preferred_element_type=jnp.float32)


def _tc1_body(x, W0, b0, preWd, preb, preWs, h_o, a_o, b_o):
    h = _dot(x[...], W0[...]) + b0[...]
    h = jnp.where(h > 0, h, 0.2 * h)
    h_o[...] = h
    a_o[...] = _dot(h, preWd[...]) + preb[...]
    b_o[...] = _dot(h, preWs[...])


def _combine(h, A, SS, SM, cnt, postW, postb, linW, linb):
    c = cnt
    mean = (c * A + SS) / jnp.maximum(c, 1.0)
    smax = jnp.where(c > 0, A + SM, 0.0)
    deg = jnp.maximum(c, 1.0)
    satt = AVG_LOG / jnp.log(deg + 1.0)
    slin = deg / AVG_LIN
    y = (_dot(h, postW[0:D]) + _dot(mean * satt, postW[D:2 * D])
         + _dot(smax * satt, postW[2 * D:3 * D])
         + _dot(mean * slin, postW[3 * D:4 * D])
         + _dot(smax * slin, postW[4 * D:5 * D]) + postb[...])
    z = _dot(y, linW[...]) + linb[...]
    return jnp.maximum(z, 0.0)


def _tc_mid_body(h, A, SS, SM, cnt, postW, postb, linW, linb,
                 preWd, preb, preWs, h_o, a_o, b_o):
    h1 = _combine(h[...], A[...], SS[...], SM[...], cnt[...],
                  postW, postb, linW, linb)
    h_o[...] = h1
    a_o[...] = _dot(h1, preWd[...]) + preb[...]
    b_o[...] = _dot(h1, preWs[...])


def _tc_final_body(h, A, SS, SM, cnt, postW, postb, linW, linb,
                   W2p, b2p, out_o):
    h2 = _combine(h[...], A[...], SS[...], SM[...], cnt[...],
                  postW, postb, linW, linb)
    out_o[...] = _dot(h2, W2p[...]) + b2p[...]


def _rb(shape):
    nd = len(shape)
    return pl.BlockSpec((ROWS,) + shape[1:],
                        lambda i, nd=nd: (i,) + (0,) * (nd - 1))


def _full(shape):
    nd = len(shape)
    return pl.BlockSpec(shape, lambda i, nd=nd: (0,) * nd)


_tc1 = pl.pallas_call(
    _tc1_body,
    grid=(GRID,),
    in_specs=[_rb((N, D)), _full((D, D)), _full((1, D)), _full((D, D)),
              _full((1, D)), _full((D, D))],
    out_specs=[_rb((N, D))] * 3,
    out_shape=[jax.ShapeDtypeStruct((N, D), jnp.float32)] * 3,
)

_tc_mid = pl.pallas_call(
    _tc_mid_body,
    grid=(GRID,),
    in_specs=[_rb((N, D)), _rb((N, D)), _rb((N, D)), _rb((N, D)),
              _rb((N, 1)), _full((5 * D, D)), _full((1, D)), _full((D, D)),
              _full((1, D)), _full((D, D)), _full((1, D)), _full((D, D))],
    out_specs=[_rb((N, D))] * 3,
    out_shape=[jax.ShapeDtypeStruct((N, D), jnp.float32)] * 3,
)

_tc_final = pl.pallas_call(
    _tc_final_body,
    grid=(GRID,),
    in_specs=[_rb((N, D)), _rb((N, D)), _rb((N, D)), _rb((N, D)),
              _rb((N, 1)), _full((5 * D, D)), _full((1, D)), _full((D, D)),
              _full((1, D)), _full((D, 8)), _full((1, 8))],
    out_specs=[_rb((N, 8))],
    out_shape=[jax.ShapeDtypeStruct((N, 8), jnp.float32)],
)


def kernel(x, adj, edge_index, W0, b0, pre_W1, pre_b1, post_W1, post_b1,
           lin_W1, lin_b1, pre_W2, pre_b2, post_W2, post_b2, lin_W2, lin_b2,
           W2, b2):
    src = edge_index[0]
    dst = edge_index[1]
    srcl, dstl, nedges = _sc_prep(src, dst)

    h0, A1, B1 = _tc1(x, W0, b0.reshape(1, D), pre_W1[:D],
                      pre_b1.reshape(1, D), pre_W1[D:])
    SSf, SMf, CNTf = _sc_layer_count(B1, srcl, dstl, nedges)
    SS1 = SSf.reshape(NPAD, D)[:N]
    SM1 = SMf.reshape(NPAD, D)[:N]
    cnt = CNTf.reshape(NPAD, 16)[:N, :1]

    h1, A2, B2 = _tc_mid(h0, A1, SS1, SM1, cnt, post_W1,
                         post_b1.reshape(1, D), lin_W1, lin_b1.reshape(1, D),
                         pre_W2[:D], pre_b2.reshape(1, D), pre_W2[D:])
    SSf2, SMf2 = _sc_layer_plain(B2, srcl, dstl, nedges)
    SS2 = SSf2.reshape(NPAD, D)[:N]
    SM2 = SMf2.reshape(NPAD, D)[:N]

    W2p = jnp.pad(W2, ((0, 0), (0, 7)))
    b2p = jnp.pad(b2, (0, 7)).reshape(1, 8)
    (out8,) = _tc_final(h1, A2, SS2, SM2, cnt, post_W2,
                        post_b2.reshape(1, D), lin_W2, lin_b2.reshape(1, D),
                        W2p, b2p)
    return out8[:, :1]


# speedup vs baseline: 3.5487x; 3.5487x over previous
import functools

import jax
import jax.numpy as jnp
from jax import lax
from jax.experimental import pallas as pl
from jax.experimental.pallas import tpu as pltpu
from jax.experimental.pallas import tpu_sc as plsc

N = 10000
E = 320000
D = 128
AVG_LIN = 32.0
AVG_LOG = 3.4965075810294595

NT = 32
NPT = 320
NPAD = NT * NPT
CAP = 12288
CE = 64
CHUNK = 8000
NCHUNKS = E // CHUNK
NGROUPS = CHUNK // 16
ACC_ROWS = NPT + 1
DUMMY = NPT
NEG = -3.0e38

_MESH = plsc.VectorSubcoreMesh(core_axis_name="c", subcore_axis_name="s")


def _wid():
    return lax.axis_index("s") * 2 + lax.axis_index("c")


def _gat(v, idx):
    return v.at[idx].get(mode="promise_in_bounds")


def _prefix_incl(m):
    p = jnp.where(m, jnp.full((16,), 1, jnp.int32),
                  jnp.zeros((16,), jnp.int32))
    idx = lax.iota(jnp.int32, 16)
    for k in (1, 2, 4, 8):
        g = _gat(p, jnp.maximum(idx - k, 0))
        p = p + jnp.where(idx >= k, g, 0)
    return p


def _compact_perm(incl):
    j = lax.iota(jnp.int32, 16)
    c = jnp.zeros((16,), jnp.int32)
    for step in (8, 4, 2, 1):
        val = _gat(incl, c + (step - 1))
        c = jnp.where(val <= j, c + step, c)
    return jnp.minimum(c, 15)


@functools.partial(
    pl.kernel,
    out_type=(
        jax.ShapeDtypeStruct((NT, CAP), jnp.int32),
        jax.ShapeDtypeStruct((NT, CAP), jnp.int32),
        jax.ShapeDtypeStruct((NT, 16), jnp.int32),
    ),
    mesh=_MESH,
    scratch_types=[
        pltpu.VMEM((CHUNK,), jnp.int32),
        pltpu.VMEM((CHUNK,), jnp.int32),
        pltpu.VMEM((CAP + CE,), jnp.int32),
        pltpu.VMEM((CAP + CE,), jnp.int32),
        pltpu.VMEM((16,), jnp.int32),
    ],
)
def _sc_prep(src_hbm, dst_hbm, srcl_hbm, dstl_hbm, nedges_hbm,
             sbuf, dbuf, srcl_v, dstl_v, nbuf):
    t = _wid()
    lo = t * NPT
    hi = lo + NPT

    def chunk_body(c, wp):
        pltpu.sync_copy(src_hbm.at[pl.ds(c * CHUNK, CHUNK)], sbuf)
        pltpu.sync_copy(dst_hbm.at[pl.ds(c * CHUNK, CHUNK)], dbuf)

        def group_body(g, wp):
            dvec = dbuf[pl.ds(g * 16, 16)]
            svec = sbuf[pl.ds(g * 16, 16)]
            m = (dvec >= lo) & (dvec < hi)
            incl = _prefix_incl(m)
            n = incl[15]
            c = _compact_perm(incl)
            w0 = jnp.minimum(wp, CAP)
            srcl_v[pl.ds(w0, 16)] = _gat(svec, c)
            dstl_v[pl.ds(w0, 16)] = _gat(dvec - lo, c)
            return wp + n

        return lax.fori_loop(0, NGROUPS, group_body, wp)

    wp = lax.fori_loop(0, NCHUNKS, chunk_body, jnp.int32(0))
    wp = jnp.minimum(wp, CAP)
    zeros16 = jnp.zeros((16,), jnp.int32)
    dummy16 = jnp.full((16,), DUMMY, jnp.int32)
    for k in range(CE // 16):
        srcl_v[pl.ds(wp + k * 16, 16)] = zeros16
        dstl_v[pl.ds(wp + k * 16, 16)] = dummy16
    wp_pad = ((wp + CE - 1) // CE) * CE
    pltpu.sync_copy(srcl_v.at[pl.ds(0, CAP)], srcl_hbm.at[t])
    pltpu.sync_copy(dstl_v.at[pl.ds(0, CAP)], dstl_hbm.at[t])
    iota = lax.iota(jnp.int32, 16)
    nbuf[...] = jnp.where(iota == 0, wp_pad, 0)
    pltpu.sync_copy(nbuf, nedges_hbm.at[t])


def _make_sc_layer(with_count):
    outs = [
        jax.ShapeDtypeStruct((NPAD * D,), jnp.float32),
        jax.ShapeDtypeStruct((NPAD * D,), jnp.float32),
    ]
    if with_count:
        outs.append(jax.ShapeDtypeStruct((NPAD * 16,), jnp.float32))

    scratch = [
        pltpu.VMEM((CAP,), jnp.int32),
        pltpu.VMEM((CAP + 16,), jnp.int32),
        pltpu.VMEM((CE, D), jnp.float32),
        pltpu.VMEM((CE, D), jnp.float32),
        pltpu.VMEM((ACC_ROWS * D,), jnp.float32),
        pltpu.VMEM((ACC_ROWS * D,), jnp.float32),
        pltpu.VMEM((ACC_ROWS * 16,), jnp.float32),
        pltpu.VMEM((16,), jnp.int32),
        pltpu.SemaphoreType.DMA,
        pltpu.SemaphoreType.DMA,
    ]

    def body(b_hbm, srcl_hbm, dstl_hbm, nedges_hbm, ss_hbm, sm_hbm, *rest):
        if with_count:
            cnt_hbm = rest[0]
            rest = rest[1:]
        srcl_v, dstl_v, r0, r1, acc_s, acc_m, acc_c, nbuf, sem0, sem1 = rest
        t = _wid()
        pltpu.sync_copy(srcl_hbm.at[t], srcl_v)
        pltpu.sync_copy(dstl_hbm.at[t], dstl_v.at[pl.ds(0, CAP)])
        pltpu.sync_copy(nedges_hbm.at[t], nbuf)
        my_n = nbuf[...][0]
        nchunks = my_n // CE

        zf = jnp.zeros((16,), jnp.float32)
        nf = jnp.full((16,), NEG, jnp.float32)

        def init_body(i, _):
            acc_s[pl.ds(i * 16, 16)] = zf
            acc_m[pl.ds(i * 16, 16)] = nf
            return 0
        lax.fori_loop(0, ACC_ROWS * D // 16, init_body, 0)
        if with_count:
            def initc_body(i, _):
                acc_c[pl.ds(i * 16, 16)] = zf
                return 0
            lax.fori_loop(0, ACC_ROWS, initc_body, 0)

        ones = jnp.ones((16,), jnp.float32)

        def process(rbuf, ebase):
            def edge_body(e, _):
                d = dstl_v[pl.ds(ebase + e, 16)][0]
                ab = d * D
                for f in range(D // 16):
                    r = rbuf[e, pl.ds(f * 16, 16)]
                    plsc.addupdate(acc_s.at[pl.ds(ab + f * 16, 16)], r)
                    mm = acc_m[pl.ds(ab + f * 16, 16)]
                    acc_m[pl.ds(ab + f * 16, 16)] = jnp.maximum(mm, r)
                if with_count:
                    plsc.addupdate(acc_c.at[pl.ds(d * 16, 16)], ones)
                return 0
            lax.fori_loop(0, CE, edge_body, 0)

        def issue(c, rbuf, sem):
            pltpu.async_copy(
                b_hbm.at[srcl_v.at[pl.ds(c * CE, CE)]], rbuf, sem)

        def wait(rbuf, sem):
            pltpu.make_async_copy(
                b_hbm.at[srcl_v.at[pl.ds(0, CE)]], rbuf, sem).wait()

        @pl.when(nchunks > 0)
        def _():
            issue(0, r0, sem0)

        def chunk_body(c, _):
            even = (c % 2) == 0

            @pl.when(even)
            def _():
                wait(r0, sem0)

                @pl.when(c + 1 < nchunks)
                def _():
                    issue(c + 1, r1, sem1)
                process(r0, c * CE)

            @pl.when(jnp.logical_not(even))
            def _():
                wait(r1, sem1)

                @pl.when(c + 1 < nchunks)
                def _():
                    issue(c + 1, r0, sem0)
                process(r1, c * CE)
            return 0

        lax.fori_loop(0, nchunks, chunk_body, 0)

        pltpu.sync_copy(acc_s.at[pl.ds(0, NPT * D)],
                        ss_hbm.at[pl.ds(t * NPT * D, NPT * D)])
        pltpu.sync_copy(acc_m.at[pl.ds(0, NPT * D)],
                        sm_hbm.at[pl.ds(t * NPT * D, NPT * D)])
        if with_count:
            pltpu.sync_copy(acc_c.at[pl.ds(0, NPT * 16)],
                            cnt_hbm.at[pl.ds(t * NPT * 16, NPT * 16)])

    return pl.kernel(body, out_type=tuple(outs), mesh=_MESH,
                     scratch_types=scratch)


_sc_layer_count = _make_sc_layer(True)
_sc_layer_plain = _make_sc_layer(False)


ROWS = 2000
GRID = N // ROWS


def _dot(a, b):
    return jnp.dot(a, b, preferred_element_type=jnp.float32)


def _tc1_body(x, W0, b0, preWd, preb, preWs, h_o, a_o, b_o):
    h = _dot(x[...], W0[...]) + b0[...]
    h = jnp.where(h > 0, h, 0.2 * h)
    h_o[...] = h
    a_o[...] = _dot(h, preWd[...]) + preb[...]
    b_o[...] = _dot(h, preWs[...])


def _combine(h, A, SS, SM, cnt, postW, postb, linW, linb):
    c = cnt
    mean = (c * A + SS) / jnp.maximum(c, 1.0)
    smax = jnp.where(c > 0, A + SM, 0.0)
    deg = jnp.maximum(c, 1.0)
    satt = AVG_LOG / jnp.log(deg + 1.0)
    slin = deg / AVG_LIN
    y = (_dot(h, postW[0:D]) + _dot(mean * satt, postW[D:2 * D])
         + _dot(smax * satt, postW[2 * D:3 * D])
         + _dot(mean * slin, postW[3 * D:4 * D])
         + _dot(smax * slin, postW[4 * D:5 * D]) + postb[...])
    z = _dot(y, linW[...]) + linb[...]
    return jnp.maximum(z, 0.0)


def _tc_mid_body(h, A, SS, SM, cnt, postW, postb, linW, linb,
                 preWd, preb, preWs, h_o, a_o, b_o):
    h1 = _combine(h[...], A[...], SS[...], SM[...], cnt[...],
                  postW, postb, linW, linb)
    h_o[...] = h1
    a_o[...] = _dot(h1, preWd[...]) + preb[...]
    b_o[...] = _dot(h1, preWs[...])


def _tc_final_body(h, A, SS, SM, cnt, postW, postb, linW, linb,
                   W2p, b2p, out_o):
    h2 = _combine(h[...], A[...], SS[...], SM[...], cnt[...],
                  postW, postb, linW, linb)
    out_o[...] = _dot(h2, W2p[...]) + b2p[...]


def _rb(shape):
    nd = len(shape)
    return pl.BlockSpec((ROWS,) + shape[1:],
                        lambda i, nd=nd: (i,) + (0,) * (nd - 1))


def _full(shape):
    nd = len(shape)
    return pl.BlockSpec(shape, lambda i, nd=nd: (0,) * nd)


_tc1 = pl.pallas_call(
    _tc1_body,
    grid=(GRID,),
    in_specs=[_rb((N, D)), _full((D, D)), _full((1, D)), _full((D, D)),
              _full((1, D)), _full((D, D))],
    out_specs=[_rb((N, D))] * 3,
    out_shape=[jax.ShapeDtypeStruct((N, D), jnp.float32)] * 3,
)

_tc_mid = pl.pallas_call(
    _tc_mid_body,
    grid=(GRID,),
    in_specs=[_rb((N, D)), _rb((N, D)), _rb((N, D)), _rb((N, D)),
              _rb((N, 1)), _full((5 * D, D)), _full((1, D)), _full((D, D)),
              _full((1, D)), _full((D, D)), _full((1, D)), _full((D, D))],
    out_specs=[_rb((N, D))] * 3,
    out_shape=[jax.ShapeDtypeStruct((N, D), jnp.float32)] * 3,
)

_tc_final = pl.pallas_call(
    _tc_final_body,
    grid=(GRID,),
    in_specs=[_rb((N, D)), _rb((N, D)), _rb((N, D)), _rb((N, D)),
              _rb((N, 1)), _full((5 * D, D)), _full((1, D)), _full((D, D)),
              _full((1, D)), _full((D, 8)), _full((1, 8))],
    out_specs=[_rb((N, 8))],
    out_shape=[jax.ShapeDtypeStruct((N, 8), jnp.float32)],
)


def kernel(x, adj, edge_index, W0, b0, pre_W1, pre_b1, post_W1, post_b1,
           lin_W1, lin_b1, pre_W2, pre_b2, post_W2, post_b2, lin_W2, lin_b2,
           W2, b2):
    src = edge_index[0]
    dst = edge_index[1]
    srcl, dstl, nedges = _sc_prep(src, dst)

    h0, A1, B1 = _tc1(x, W0, b0.reshape(1, D), pre_W1[:D],
                      pre_b1.reshape(1, D), pre_W1[D:])
    SSf, SMf, CNTf = _sc_layer_count(B1, srcl, dstl, nedges)
    SS1 = SSf.reshape(NPAD, D)[:N]
    SM1 = SMf.reshape(NPAD, D)[:N]
    cnt = CNTf.reshape(NPAD, 16)[:N, :1]

    h1, A2, B2 = _tc_mid(h0, A1, SS1, SM1, cnt, post_W1,
                         post_b1.reshape(1, D), lin_W1, lin_b1.reshape(1, D),
                         pre_W2[:D], pre_b2.reshape(1, D), pre_W2[D:])
    SSf2, SMf2 = _sc_layer_plain(B2, srcl, dstl, nedges)
    SS2 = SSf2.reshape(NPAD, D)[:N]
    SM2 = SMf2.reshape(NPAD, D)[:N]

    W2p = jnp.pad(W2, ((0, 0), (0, 7)))
    b2p = jnp.pad(b2, (0, 7)).reshape(1, 8)
    (out8,) = _tc_final(h1, A2, SS2, SM2, cnt, post_W2,
                        post_b2.reshape(1, D), lin_W2, lin_b2.reshape(1, D),
                        W2p, b2p)
    return out8[:, :1]


# speedup vs baseline: 4.6791x; 1.3186x over previous
import functools

import jax
import jax.numpy as jnp
from jax import lax
from jax.experimental import pallas as pl
from jax.experimental.pallas import tpu as pltpu
from jax.experimental.pallas import tpu_sc as plsc

N = 10000
E = 320000
D = 128
AVG_LIN = 32.0
AVG_LOG = 3.4965075810294595

NT = 32
NPT = 320
NPAD = NT * NPT
CAP = 12288
CE = 64
SCAN = E // NT
BCAP = 672
BSTRIDE = BCAP + 16
PBUF = CAP + BSTRIDE + CE
ACC_ROWS = NPT + 1
DUMMY = NPT
NEG = -3.0e38
DIV320_M = 104858
DIV320_S = 25

_MESH = plsc.VectorSubcoreMesh(core_axis_name="c", subcore_axis_name="s")


def _wid():
    return lax.axis_index("s") * 2 + lax.axis_index("c")


def _gat(v, idx):
    return v.at[idx].get(mode="promise_in_bounds")


@functools.partial(
    pl.kernel,
    out_type=(
        jax.ShapeDtypeStruct((NT * 32 * BSTRIDE,), jnp.int32),
        jax.ShapeDtypeStruct((NT * 512,), jnp.int32),
    ),
    mesh=_MESH,
    scratch_types=[
        pltpu.VMEM((SCAN,), jnp.int32),
        pltpu.VMEM((SCAN,), jnp.int32),
        pltpu.VMEM((32 * BSTRIDE,), jnp.int32),
        pltpu.VMEM((512,), jnp.int32),
    ],
)
def _sc_bucket(src_hbm, dst_hbm, pk_hbm, cnt_hbm, sbuf, dbuf, bk, wpv):
    t = _wid()
    pltpu.sync_copy(src_hbm.at[pl.ds(t * SCAN, SCAN)], sbuf)
    pltpu.sync_copy(dst_hbm.at[pl.ds(t * SCAN, SCAN)], dbuf)
    zeros16 = jnp.zeros((16,), jnp.int32)

    def initb(i, _):
        wpv[pl.ds(i * 16, 16)] = zeros16
        return 0
    lax.fori_loop(0, 32, initb, 0)

    iota = lax.iota(jnp.int32, 16)

    def group(g, _):
        dvec = dbuf[pl.ds(g * 16, 16)]
        svec = sbuf[pl.ds(g * 16, 16)]
        bvec = (dvec * DIV320_M) >> DIV320_S
        pvec = svec * 512 + (dvec - bvec * NPT)
        for i in range(16):
            b = bvec[i]
            w = wpv[pl.ds(b * 16, 16)][0]
            wc = jnp.minimum(w, BCAP - 1)
            wpv[pl.ds(b * 16, 16)] = zeros16 + (wc + 1)
            rot = _gat(pvec, (iota + i) & 15)
            bk[pl.ds(b * BSTRIDE + wc, 16)] = rot
        return 0
    lax.fori_loop(0, SCAN // 16, group, 0)

    pltpu.sync_copy(bk, pk_hbm.at[pl.ds(t * 32 * BSTRIDE, 32 * BSTRIDE)])
    pltpu.sync_copy(wpv, cnt_hbm.at[pl.ds(t * 512, 512)])


@functools.partial(
    pl.kernel,
    out_type=(
        jax.ShapeDtypeStruct((NT, CAP), jnp.int32),
        jax.ShapeDtypeStruct((NT, CAP), jnp.int32),
        jax.ShapeDtypeStruct((NT, 16), jnp.int32),
    ),
    mesh=_MESH,
    scratch_types=[
        pltpu.VMEM((PBUF,), jnp.int32),
        pltpu.VMEM((NT * 512,), jnp.int32),
        pltpu.VMEM((CAP,), jnp.int32),
        pltpu.VMEM((CAP,), jnp.int32),
        pltpu.VMEM((16,), jnp.int32),
    ],
)
def _sc_concat(pk_hbm, cnt_hbm, srcl_hbm, dstl_hbm, nedges_hbm,
               pbuf, cntv, sv, dv, nbuf):
    t = _wid()
    pltpu.sync_copy(cnt_hbm, cntv)

    dummy16 = jnp.full((16,), DUMMY, jnp.int32)

    def seg(s, wp8):
        pltpu.sync_copy(
            pk_hbm.at[pl.ds(s * 32 * BSTRIDE + t * BSTRIDE, BSTRIDE)],
            pbuf.at[pl.ds(wp8 * 8, BSTRIDE)])
        c = cntv[pl.ds(s * 512 + t * 16, 16)][0]
        pbuf[pl.ds(wp8 * 8 + c, 16)] = dummy16
        return jnp.minimum(wp8 + ((c + 7) >> 3), CAP // 8)
    wp = lax.fori_loop(0, NT, seg, jnp.int32(0)) * 8

    for k in range(CE // 16):
        pbuf[pl.ds(wp + k * 16, 16)] = dummy16
    wp_pad = ((wp + CE - 1) // CE) * CE

    def ug(g, _):
        pv = pbuf[pl.ds(g * 16, 16)]
        sv[pl.ds(g * 16, 16)] = pv >> 9
        dv[pl.ds(g * 16, 16)] = pv & 511
        return 0
    lax.fori_loop(0, CAP // 16, ug, 0)

    pltpu.sync_copy(sv, srcl_hbm.at[t])
    pltpu.sync_copy(dv, dstl_hbm.at[t])
    iota = lax.iota(jnp.int32, 16)
    nbuf[...] = jnp.where(iota == 0, wp_pad, 0)
    pltpu.sync_copy(nbuf, nedges_hbm.at[t])


def _make_sc_layer(with_count):
    outs = [
        jax.ShapeDtypeStruct((NPAD * D,), jnp.float32),
        jax.ShapeDtypeStruct((NPAD * D,), jnp.float32),
    ]
    if with_count:
        outs.append(jax.ShapeDtypeStruct((NPAD * 16,), jnp.float32))

    scratch = [
        pltpu.VMEM((CAP,), jnp.int32),
        pltpu.VMEM((CAP + 16,), jnp.int32),
        pltpu.VMEM((CE, D), jnp.float32),
        pltpu.VMEM((CE, D), jnp.float32),
        pltpu.VMEM((ACC_ROWS * D,), jnp.float32),
        pltpu.VMEM((ACC_ROWS * D,), jnp.float32),
        pltpu.VMEM((ACC_ROWS * 16,), jnp.float32),
        pltpu.VMEM((16,), jnp.int32),
        pltpu.SemaphoreType.DMA,
        pltpu.SemaphoreType.DMA,
    ]

    def body(b_hbm, srcl_hbm, dstl_hbm, nedges_hbm, ss_hbm, sm_hbm, *rest):
        if with_count:
            cnt_hbm = rest[0]
            rest = rest[1:]
        srcl_v, dstl_v, r0, r1, acc_s, acc_m, acc_c, nbuf, sem0, sem1 = rest
        t = _wid()
        pltpu.sync_copy(srcl_hbm.at[t], srcl_v)
        pltpu.sync_copy(dstl_hbm.at[t], dstl_v.at[pl.ds(0, CAP)])
        pltpu.sync_copy(nedges_hbm.at[t], nbuf)
        my_n = nbuf[...][0]
        nchunks = my_n // CE

        zf = jnp.zeros((16,), jnp.float32)
        nf = jnp.full((16,), NEG, jnp.float32)

        def init_body(i, _):
            acc_s[pl.ds(i * 16, 16)] = zf
            acc_m[pl.ds(i * 16, 16)] = nf
            return 0
        lax.fori_loop(0, ACC_ROWS * D // 16, init_body, 0)
        if with_count:
            def initc_body(i, _):
                acc_c[pl.ds(i * 16, 16)] = zf
                return 0
            lax.fori_loop(0, ACC_ROWS, initc_body, 0)

        ones = jnp.ones((16,), jnp.float32)

        def process(rbuf, ebase):
            def edge_body(e, _):
                d = dstl_v[pl.ds(ebase + e, 16)][0]
                ab = d * D
                for f in range(D // 16):
                    r = rbuf[e, pl.ds(f * 16, 16)]
                    plsc.addupdate(acc_s.at[pl.ds(ab + f * 16, 16)], r)
                    mm = acc_m[pl.ds(ab + f * 16, 16)]
                    acc_m[pl.ds(ab + f * 16, 16)] = jnp.maximum(mm, r)
                if with_count:
                    plsc.addupdate(acc_c.at[pl.ds(d * 16, 16)], ones)
                return 0
            lax.fori_loop(0, CE, edge_body, 0)

        def issue(c, rbuf, sem):
            pltpu.async_copy(
                b_hbm.at[srcl_v.at[pl.ds(c * CE, CE)]], rbuf, sem)

        def wait(rbuf, sem):
            pltpu.make_async_copy(
                b_hbm.at[srcl_v.at[pl.ds(0, CE)]], rbuf, sem).wait()

        @pl.when(nchunks > 0)
        def _():
            issue(0, r0, sem0)

        def chunk_body(c, _):
            even = (c % 2) == 0

            @pl.when(even)
            def _():
                wait(r0, sem0)

                @pl.when(c + 1 < nchunks)
                def _():
                    issue(c + 1, r1, sem1)
                process(r0, c * CE)

            @pl.when(jnp.logical_not(even))
            def _():
                wait(r1, sem1)

                @pl.when(c + 1 < nchunks)
                def _():
                    issue(c + 1, r0, sem0)
                process(r1, c * CE)
            return 0

        lax.fori_loop(0, nchunks, chunk_body, 0)

        pltpu.sync_copy(acc_s.at[pl.ds(0, NPT * D)],
                        ss_hbm.at[pl.ds(t * NPT * D, NPT * D)])
        pltpu.sync_copy(acc_m.at[pl.ds(0, NPT * D)],
                        sm_hbm.at[pl.ds(t * NPT * D, NPT * D)])
        if with_count:
            pltpu.sync_copy(acc_c.at[pl.ds(0, NPT * 16)],
                            cnt_hbm.at[pl.ds(t * NPT * 16, NPT * 16)])

    return pl.kernel(body, out_type=tuple(outs), mesh=_MESH,
                     scratch_types=scratch)


_sc_layer_count = _make_sc_layer(True)
_sc_layer_plain = _make_sc_layer(False)


ROWS = 2000
GRID = N // ROWS


def _dot(a, b):
    return jnp.dot(a, b, preferred_element_type=jnp.float32)


def _tc1_body(x, W0, b0, preWd, preb, preWs, h_o, a_o, b_o):
    h = _dot(x[...], W0[...]) + b0[...]
    h = jnp.where(h > 0, h, 0.2 * h)
    h_o[...] = h
    a_o[...] = _dot(h, preWd[...]) + preb[...]
    b_o[...] = _dot(h, preWs[...])


def _combine(h, A, SS, SM, cnt, postW, postb, linW, linb):
    c = cnt
    mean = (c * A + SS) / jnp.maximum(c, 1.0)
    smax = jnp.where(c > 0, A + SM, 0.0)
    deg = jnp.maximum(c, 1.0)
    satt = AVG_LOG / jnp.log(deg + 1.0)
    slin = deg / AVG_LIN
    y = (_dot(h, postW[0:D]) + _dot(mean * satt, postW[D:2 * D])
         + _dot(smax * satt, postW[2 * D:3 * D])
         + _dot(mean * slin, postW[3 * D:4 * D])
         + _dot(smax * slin, postW[4 * D:5 * D]) + postb[...])
    z = _dot(y, linW[...]) + linb[...]
    return jnp.maximum(z, 0.0)


def _tc_mid_body(h, A, SS, SM, cnt, postW, postb, linW, linb,
                 preWd, preb, preWs, h_o, a_o, b_o):
    h1 = _combine(h[...], A[...], SS[...], SM[...], cnt[...],
                  postW, postb, linW, linb)
    h_o[...] = h1
    a_o[...] = _dot(h1, preWd[...]) + preb[...]
    b_o[...] = _dot(h1, preWs[...])


def _tc_final_body(h, A, SS, SM, cnt, postW, postb, linW, linb,
                   W2p, b2p, out_o):
    h2 = _combine(h[...], A[...], SS[...], SM[...], cnt[...],
                  postW, postb, linW, linb)
    out_o[...] = _dot(h2, W2p[...]) + b2p[...]


def _rb(shape):
    nd = len(shape)
    return pl.BlockSpec((ROWS,) + shape[1:],
                        lambda i, nd=nd: (i,) + (0,) * (nd - 1))


def _full(shape):
    nd = len(shape)
    return pl.BlockSpec(shape, lambda i, nd=nd: (0,) * nd)


_tc1 = pl.pallas_call(
    _tc1_body,
    grid=(GRID,),
    in_specs=[_rb((N, D)), _full((D, D)), _full((1, D)), _full((D, D)),
              _full((1, D)), _full((D, D))],
    out_specs=[_rb((N, D))] * 3,
    out_shape=[jax.ShapeDtypeStruct((N, D), jnp.float32)] * 3,
)

_tc_mid = pl.pallas_call(
    _tc_mid_body,
    grid=(GRID,),
    in_specs=[_rb((N, D)), _rb((N, D)), _rb((N, D)), _rb((N, D)),
              _rb((N, 1)), _full((5 * D, D)), _full((1, D)), _full((D, D)),
              _full((1, D)), _full((D, D)), _full((1, D)), _full((D, D))],
    out_specs=[_rb((N, D))] * 3,
    out_shape=[jax.ShapeDtypeStruct((N, D), jnp.float32)] * 3,
)

_tc_final = pl.pallas_call(
    _tc_final_body,
    grid=(GRID,),
    in_specs=[_rb((N, D)), _rb((N, D)), _rb((N, D)), _rb((N, D)),
              _rb((N, 1)), _full((5 * D, D)), _full((1, D)), _full((D, D)),
              _full((1, D)), _full((D, 8)), _full((1, 8))],
    out_specs=[_rb((N, 8))],
    out_shape=[jax.ShapeDtypeStruct((N, 8), jnp.float32)],
)


def kernel(x, adj, edge_index, W0, b0, pre_W1, pre_b1, post_W1, post_b1,
           lin_W1, lin_b1, pre_W2, pre_b2, post_W2, post_b2, lin_W2, lin_b2,
           W2, b2):
    src = edge_index[0]
    dst = edge_index[1]
    pk, cnt = _sc_bucket(src, dst)
    srcl, dstl, nedges = _sc_concat(pk, cnt)

    h0, A1, B1 = _tc1(x, W0, b0.reshape(1, D), pre_W1[:D],
                      pre_b1.reshape(1, D), pre_W1[D:])
    SSf, SMf, CNTf = _sc_layer_count(B1, srcl, dstl, nedges)
    SS1 = SSf.reshape(NPAD, D)[:N]
    SM1 = SMf.reshape(NPAD, D)[:N]
    cnt = CNTf.reshape(NPAD, 16)[:N, :1]

    h1, A2, B2 = _tc_mid(h0, A1, SS1, SM1, cnt, post_W1,
                         post_b1.reshape(1, D), lin_W1, lin_b1.reshape(1, D),
                         pre_W2[:D], pre_b2.reshape(1, D), pre_W2[D:])
    SSf2, SMf2 = _sc_layer_plain(B2, srcl, dstl, nedges)
    SS2 = SSf2.reshape(NPAD, D)[:N]
    SM2 = SMf2.reshape(NPAD, D)[:N]

    W2p = jnp.pad(W2, ((0, 0), (0, 7)))
    b2p = jnp.pad(b2, (0, 7)).reshape(1, 8)
    (out8,) = _tc_final(h1, A2, SS2, SM2, cnt, post_W2,
                        post_b2.reshape(1, D), lin_W2, lin_b2.reshape(1, D),
                        W2p, b2p)
    return out8[:, :1]


# speedup vs baseline: 5.7716x; 1.2335x over previous
import functools

import jax
import jax.numpy as jnp
from jax import lax
from jax.experimental import pallas as pl
from jax.experimental.pallas import tpu as pltpu
from jax.experimental.pallas import tpu_sc as plsc

N = 10000
E = 320000
D = 128
AVG_LIN = 32.0
AVG_LOG = 3.4965075810294595

NT = 32
NPT = 320
NPAD = NT * NPT
CAP = 12288
CE = 64
SCAN = E // NT
BCAP = 672
BSTRIDE = BCAP + 16
PBUF = CAP + BSTRIDE + CE
BINCAP = 96
BINSTRIDE = BINCAP + 16
NGR = CAP // 8
ACC_ROWS = NPT + 1
DUMMY = NPT
NEG = -3.0e38
DIV320_M = 104858
DIV320_S = 25

_MESH = plsc.VectorSubcoreMesh(core_axis_name="c", subcore_axis_name="s")


def _wid():
    return lax.axis_index("s") * 2 + lax.axis_index("c")


def _gat(v, idx):
    return v.at[idx].get(mode="promise_in_bounds")


@functools.partial(
    pl.kernel,
    out_type=(
        jax.ShapeDtypeStruct((NT * 32 * BSTRIDE,), jnp.int32),
        jax.ShapeDtypeStruct((NT * 512,), jnp.int32),
    ),
    mesh=_MESH,
    scratch_types=[
        pltpu.VMEM((SCAN,), jnp.int32),
        pltpu.VMEM((SCAN,), jnp.int32),
        pltpu.VMEM((32 * BSTRIDE,), jnp.int32),
        pltpu.VMEM((512,), jnp.int32),
    ],
)
def _sc_bucket(src_hbm, dst_hbm, pk_hbm, cnt_hbm, sbuf, dbuf, bk, wpv):
    t = _wid()
    pltpu.sync_copy(src_hbm.at[pl.ds(t * SCAN, SCAN)], sbuf)
    pltpu.sync_copy(dst_hbm.at[pl.ds(t * SCAN, SCAN)], dbuf)
    zeros16 = jnp.zeros((16,), jnp.int32)

    def initb(i, _):
        wpv[pl.ds(i * 16, 16)] = zeros16
        return 0
    lax.fori_loop(0, 32, initb, 0)

    iota = lax.iota(jnp.int32, 16)

    def group(g, _):
        dvec = dbuf[pl.ds(g * 16, 16)]
        svec = sbuf[pl.ds(g * 16, 16)]
        bvec = (dvec * DIV320_M) >> DIV320_S
        pvec = svec * 512 + (dvec - bvec * NPT)
        for i in range(16):
            b = bvec[i]
            w = wpv[pl.ds(b * 16, 16)][0]
            wc = jnp.minimum(w, BCAP - 1)
            wpv[pl.ds(b * 16, 16)] = zeros16 + (wc + 1)
            rot = _gat(pvec, (iota + i) & 15)
            bk[pl.ds(b * BSTRIDE + wc, 16)] = rot
        return 0
    lax.fori_loop(0, SCAN // 16, group, 0)

    pltpu.sync_copy(bk, pk_hbm.at[pl.ds(t * 32 * BSTRIDE, 32 * BSTRIDE)])
    pltpu.sync_copy(wpv, cnt_hbm.at[pl.ds(t * 512, 512)])


@functools.partial(
    pl.kernel,
    out_type=(
        jax.ShapeDtypeStruct((NT, CAP), jnp.int32),
        jax.ShapeDtypeStruct((NT, CAP), jnp.int32),
        jax.ShapeDtypeStruct((NT, 16), jnp.int32),
        jax.ShapeDtypeStruct((NT, NGR), jnp.int32),
        jax.ShapeDtypeStruct((NT, NPT * 16), jnp.int32),
    ),
    mesh=_MESH,
    scratch_types=[
        pltpu.VMEM((BSTRIDE,), jnp.int32),
        pltpu.VMEM((NT * 512,), jnp.int32),
        pltpu.VMEM((NPT * BINSTRIDE,), jnp.int32),
        pltpu.VMEM((NPT * 16,), jnp.int32),
        pltpu.VMEM((CAP + BINCAP,), jnp.int32),
        pltpu.VMEM((CAP + BINCAP,), jnp.int32),
        pltpu.VMEM((NGR + 16,), jnp.int32),
        pltpu.VMEM((16,), jnp.int32),
    ],
)
def _sc_concat(pk_hbm, cnt_hbm, srcl_hbm, dstl_hbm, nedges_hbm, aux_hbm,
               deg_hbm, pbuf, cntv, bins, wpv, sv, dv, av, nbuf):
    t = _wid()
    pltpu.sync_copy(cnt_hbm, cntv)
    zeros16 = jnp.zeros((16,), jnp.int32)
    iota = lax.iota(jnp.int32, 16)

    def initw(i, _):
        wpv[pl.ds(i * 16, 16)] = zeros16
        return 0
    lax.fori_loop(0, NPT, initw, 0)

    def inita(i, _):
        av[pl.ds(i * 16, 16)] = zeros16
        return 0
    lax.fori_loop(0, (NGR + 16) // 16, inita, 0)

    def seg(s, _):
        pltpu.sync_copy(
            pk_hbm.at[pl.ds(s * 32 * BSTRIDE + t * BSTRIDE, BSTRIDE)], pbuf)
        c = cntv[pl.ds(s * 512 + t * 16, 16)][0]

        def edge(e, _):
            pv = pbuf[pl.ds(e, 16)][0]
            d = pv & 511
            w = wpv[pl.ds(d * 16, 16)][0]
            wc = jnp.minimum(w, BINCAP - 1)
            wpv[pl.ds(d * 16, 16)] = zeros16 + (wc + 1)
            bins[pl.ds(d * BINSTRIDE + wc, 16)] = zeros16 + pv
            return 0
        lax.fori_loop(0, c, edge, 0)
        return 0
    lax.fori_loop(0, NT, seg, 0)

    def padbin(d, _):
        c = wpv[pl.ds(d * 16, 16)][0]
        last = bins[pl.ds(d * BINSTRIDE + jnp.maximum(c, 1) - 1, 16)][0]
        bins[pl.ds(d * BINSTRIDE + c, 16)] = zeros16 + last
        return 0
    lax.fori_loop(0, NPT, padbin, 0)

    def cat(d, wp8):
        c = wpv[pl.ds(d * 16, 16)][0]
        c8 = (c + 7) & ~7
        base = d * BINSTRIDE
        for k in range(BINCAP // 16):
            pv = bins[pl.ds(base + k * 16, 16)]
            sv[pl.ds(wp8 * 8 + k * 16, 16)] = pv >> 9
            dv[pl.ds(wp8 * 8 + k * 16, 16)] = pv & 511
        ng = c8 >> 3
        av[pl.ds(wp8, 16)] = jnp.where(iota == ng - 1, c8 - c, 0)
        return jnp.minimum(wp8 + ng, NGR)
    wp8 = lax.fori_loop(0, NPT, cat, jnp.int32(0))
    wp = wp8 * 8

    dummy16 = jnp.full((16,), DUMMY, jnp.int32)
    for k in range(CE // 16):
        sv[pl.ds(wp + k * 16, 16)] = zeros16
        dv[pl.ds(wp + k * 16, 16)] = dummy16
    wp_pad = ((wp + CE - 1) // CE) * CE

    pltpu.sync_copy(sv.at[pl.ds(0, CAP)], srcl_hbm.at[t])
    pltpu.sync_copy(dv.at[pl.ds(0, CAP)], dstl_hbm.at[t])
    pltpu.sync_copy(av.at[pl.ds(0, NGR)], aux_hbm.at[t])
    pltpu.sync_copy(wpv, deg_hbm.at[t])
    nbuf[...] = jnp.where(iota == 0, wp_pad, 0)
    pltpu.sync_copy(nbuf, nedges_hbm.at[t])


@functools.partial(
    pl.kernel,
    out_type=(
        jax.ShapeDtypeStruct((NPAD * D,), jnp.float32),
        jax.ShapeDtypeStruct((NPAD * D,), jnp.float32),
    ),
    mesh=_MESH,
    scratch_types=[
        pltpu.VMEM((CAP,), jnp.int32),
        pltpu.VMEM((CAP + 16,), jnp.int32),
        pltpu.VMEM((NGR + 16,), jnp.int32),
        pltpu.VMEM((CE, D), jnp.float32),
        pltpu.VMEM((CE, D), jnp.float32),
        pltpu.VMEM((ACC_ROWS * D,), jnp.float32),
        pltpu.VMEM((ACC_ROWS * D,), jnp.float32),
        pltpu.VMEM((16,), jnp.int32),
        pltpu.SemaphoreType.DMA,
        pltpu.SemaphoreType.DMA,
    ],
)
def _sc_layer(b_hbm, srcl_hbm, dstl_hbm, nedges_hbm, aux_hbm, ss_hbm, sm_hbm,
              srcl_v, dstl_v, aux_v, r0, r1, acc_s, acc_m, nbuf, sem0, sem1):
    t = _wid()
    pltpu.sync_copy(srcl_hbm.at[t], srcl_v)
    pltpu.sync_copy(dstl_hbm.at[t], dstl_v.at[pl.ds(0, CAP)])
    pltpu.sync_copy(aux_hbm.at[t], aux_v.at[pl.ds(0, NGR)])
    pltpu.sync_copy(nedges_hbm.at[t], nbuf)
    my_n = nbuf[...][0]
    nchunks = my_n // CE

    zf = jnp.zeros((16,), jnp.float32)
    nf = jnp.full((16,), NEG, jnp.float32)
    zi = jnp.zeros((16,), jnp.int32)

    def init_body(i, _):
        acc_s[pl.ds(i * 16, 16)] = zf
        acc_m[pl.ds(i * 16, 16)] = nf
        return 0
    lax.fori_loop(0, ACC_ROWS * D // 16, init_body, 0)

    def process(rbuf, c):
        def group_body(gg, _):
            d = dstl_v[pl.ds(c * CE + gg * 8, 16)][0]
            npad = aux_v[pl.ds(c * 8 + gg, 16)][0]
            npf = (zi + npad).astype(jnp.float32)
            ab = d * D
            for f in range(D // 16):
                v = rbuf[gg * 8, pl.ds(f * 16, 16)]
                s = v
                m = v
                for e in range(1, 8):
                    v = rbuf[gg * 8 + e, pl.ds(f * 16, 16)]
                    s = s + v
                    m = jnp.maximum(m, v)
                s = s - npf * v
                plsc.addupdate(acc_s.at[pl.ds(ab + f * 16, 16)], s)
                mm = acc_m[pl.ds(ab + f * 16, 16)]
                acc_m[pl.ds(ab + f * 16, 16)] = jnp.maximum(mm, m)
            return 0
        lax.fori_loop(0, CE // 8, group_body, 0)

    def issue(c, rbuf, sem):
        pltpu.async_copy(
            b_hbm.at[srcl_v.at[pl.ds(c * CE, CE)]], rbuf, sem)

    def wait(rbuf, sem):
        pltpu.make_async_copy(
            b_hbm.at[srcl_v.at[pl.ds(0, CE)]], rbuf, sem).wait()

    @pl.when(nchunks > 0)
    def _():
        issue(0, r0, sem0)

    def chunk_body(c, _):
        even = (c % 2) == 0

        @pl.when(even)
        def _():
            wait(r0, sem0)

            @pl.when(c + 1 < nchunks)
            def _():
                issue(c + 1, r1, sem1)
            process(r0, c)

        @pl.when(jnp.logical_not(even))
        def _():
            wait(r1, sem1)

            @pl.when(c + 1 < nchunks)
            def _():
                issue(c + 1, r0, sem0)
            process(r1, c)
        return 0

    lax.fori_loop(0, nchunks, chunk_body, 0)

    pltpu.sync_copy(acc_s.at[pl.ds(0, NPT * D)],
                    ss_hbm.at[pl.ds(t * NPT * D, NPT * D)])
    pltpu.sync_copy(acc_m.at[pl.ds(0, NPT * D)],
                    sm_hbm.at[pl.ds(t * NPT * D, NPT * D)])


ROWS = 2000
GRID = N // ROWS


def _dot(a, b):
    return jnp.dot(a, b, preferred_element_type=jnp.float32)


def _tc1_body(x, W0, b0, preWd, preb, preWs, h_o, a_o, b_o):
    h = _dot(x[...], W0[...]) + b0[...]
    h = jnp.where(h > 0, h, 0.2 * h)
    h_o[...] = h
    a_o[...] = _dot(h, preWd[...]) + preb[...]
    b_o[...] = _dot(h, preWs[...])


def _combine(h, A, SS, SM, cnt, postW, postb, linW, linb):
    c = cnt
    mean = (c * A + SS) / jnp.maximum(c, 1.0)
    smax = jnp.where(c > 0, A + SM, 0.0)
    deg = jnp.maximum(c, 1.0)
    satt = AVG_LOG / jnp.log(deg + 1.0)
    slin = deg / AVG_LIN
    y = (_dot(h, postW[0:D]) + _dot(mean * satt, postW[D:2 * D])
         + _dot(smax * satt, postW[2 * D:3 * D])
         + _dot(mean * slin, postW[3 * D:4 * D])
         + _dot(smax * slin, postW[4 * D:5 * D]) + postb[...])
    z = _dot(y, linW[...]) + linb[...]
    return jnp.maximum(z, 0.0)


def _tc_mid_body(h, A, SS, SM, cnt, postW, postb, linW, linb,
                 preWd, preb, preWs, h_o, a_o, b_o):
    h1 = _combine(h[...], A[...], SS[...], SM[...], cnt[...],
                  postW, postb, linW, linb)
    h_o[...] = h1
    a_o[...] = _dot(h1, preWd[...]) + preb[...]
    b_o[...] = _dot(h1, preWs[...])


def _tc_final_body(h, A, SS, SM, cnt, postW, postb, linW, linb,
                   W2p, b2p, out_o):
    h2 = _combine(h[...], A[...], SS[...], SM[...], cnt[...],
                  postW, postb, linW, linb)
    out_o[...] = _dot(h2, W2p[...]) + b2p[...]


def _rb(shape):
    nd = len(shape)
    return pl.BlockSpec((ROWS,) + shape[1:],
                        lambda i, nd=nd: (i,) + (0,) * (nd - 1))


def _full(shape):
    nd = len(shape)
    return pl.BlockSpec(shape, lambda i, nd=nd: (0,) * nd)


_tc1 = pl.pallas_call(
    _tc1_body,
    grid=(GRID,),
    in_specs=[_rb((N, D)), _full((D, D)), _full((1, D)), _full((D, D)),
              _full((1, D)), _full((D, D))],
    out_specs=[_rb((N, D))] * 3,
    out_shape=[jax.ShapeDtypeStruct((N, D), jnp.float32)] * 3,
)

_tc_mid = pl.pallas_call(
    _tc_mid_body,
    grid=(GRID,),
    in_specs=[_rb((N, D)), _rb((N, D)), _rb((N, D)), _rb((N, D)),
              _rb((N, 1)), _full((5 * D, D)), _full((1, D)), _full((D, D)),
              _full((1, D)), _full((D, D)), _full((1, D)), _full((D, D))],
    out_specs=[_rb((N, D))] * 3,
    out_shape=[jax.ShapeDtypeStruct((N, D), jnp.float32)] * 3,
)

_tc_final = pl.pallas_call(
    _tc_final_body,
    grid=(GRID,),
    in_specs=[_rb((N, D)), _rb((N, D)), _rb((N, D)), _rb((N, D)),
              _rb((N, 1)), _full((5 * D, D)), _full((1, D)), _full((D, D)),
              _full((1, D)), _full((D, 8)), _full((1, 8))],
    out_specs=[_rb((N, 8))],
    out_shape=[jax.ShapeDtypeStruct((N, 8), jnp.float32)],
)


def kernel(x, adj, edge_index, W0, b0, pre_W1, pre_b1, post_W1, post_b1,
           lin_W1, lin_b1, pre_W2, pre_b2, post_W2, post_b2, lin_W2, lin_b2,
           W2, b2):
    src = edge_index[0]
    dst = edge_index[1]
    pk, pkc = _sc_bucket(src, dst)
    srcl, dstl, nedges, aux, deg = _sc_concat(pk, pkc)

    h0, A1, B1 = _tc1(x, W0, b0.reshape(1, D), pre_W1[:D],
                      pre_b1.reshape(1, D), pre_W1[D:])
    SSf, SMf = _sc_layer(B1, srcl, dstl, nedges, aux)
    SS1 = SSf.reshape(NPAD, D)[:N]
    SM1 = SMf.reshape(NPAD, D)[:N]
    cnt = deg.reshape(NPAD, 16)[:N, :1].astype(jnp.float32)

    h1, A2, B2 = _tc_mid(h0, A1, SS1, SM1, cnt, post_W1,
                         post_b1.reshape(1, D), lin_W1, lin_b1.reshape(1, D),
                         pre_W2[:D], pre_b2.reshape(1, D), pre_W2[D:])
    SSf2, SMf2 = _sc_layer(B2, srcl, dstl, nedges, aux)
    SS2 = SSf2.reshape(NPAD, D)[:N]
    SM2 = SMf2.reshape(NPAD, D)[:N]

    W2p = jnp.pad(W2, ((0, 0), (0, 7)))
    b2p = jnp.pad(b2, (0, 7)).reshape(1, 8)
    (out8,) = _tc_final(h1, A2, SS2, SM2, cnt, post_W2,
                        post_b2.reshape(1, D), lin_W2, lin_b2.reshape(1, D),
                        W2p, b2p)
    return out8[:, :1]
